# Initial kernel scaffold; baseline (speedup 1.0000x reference)
import jax
import jax.numpy as jnp
from jax.experimental import pallas as pl


def kernel(data, iteration):
    raise NotImplementedError("write your pallas kernel here")


# speedup vs baseline: 14.4388x; 14.4388x over previous
import numpy as np
import jax
import jax.numpy as jnp
from jax.experimental import pallas as pl
from jax.experimental.pallas import tpu as pltpu

_N = 4096
_D = 64
_K = 512
_RB = 512
_NB = _N // _RB

_PREC = jax.lax.Precision.HIGHEST


def _init_centroid_ids():
    rng = np.random.default_rng(0)
    return np.asarray(rng.choice(_N, size=_K, replace=False))


def _kmeans_kernel(it_ref, data_ref, c0_ref, oh_ref, cent_ref):
    iota_k = jax.lax.broadcasted_iota(jnp.int32, (_RB, _K), 1)

    def assign_block(b, c, csq):
        x = data_ref[pl.ds(b * _RB, _RB), :]
        xsq = jnp.sum(x * x, axis=1, keepdims=True)
        xc = jax.lax.dot_general(
            x, c, (((1,), (1,)), ((), ())),
            preferred_element_type=jnp.float32, precision=_PREC)
        d2 = (xsq - 2.0 * xc) + csq
        m = jnp.min(d2, axis=1, keepdims=True)
        idx = jnp.min(jnp.where(d2 == m, iota_k, _K), axis=1, keepdims=True)
        oh = (iota_k == idx).astype(jnp.float32)
        return x, oh

    def stats_pass(c):
        csq = jnp.sum(c * c, axis=1)[None, :]

        def blk(b, carry):
            acc, cnt = carry
            x, oh = assign_block(b, c, csq)
            acc = acc + jax.lax.dot_general(
                oh, x, (((0,), (0,)), ((), ())),
                preferred_element_type=jnp.float32, precision=_PREC)
            cnt = cnt + jnp.sum(oh, axis=0)
            return acc, cnt

        acc0 = jnp.zeros((_K, _D), jnp.float32)
        cnt0 = jnp.zeros((_K,), jnp.float32)
        return jax.lax.fori_loop(0, _NB, blk, (acc0, cnt0))

    def iter_body(_, c):
        acc, cnt = stats_pass(c)
        return acc / cnt[:, None]

    c_final = jax.lax.fori_loop(0, it_ref[0], iter_body, c0_ref[...])
    cent_ref[...] = c_final

    csq = jnp.sum(c_final * c_final, axis=1)[None, :]

    def final_blk(b, _):
        _, oh = assign_block(b, c_final, csq)
        oh_ref[pl.ds(b * _RB, _RB), :] = oh
        return 0

    jax.lax.fori_loop(0, _NB, final_blk, 0)


def kernel(data, iteration):
    c0 = jnp.take(data, jnp.asarray(_init_centroid_ids()), axis=0)
    it = jnp.asarray(iteration, jnp.int32).reshape(1)
    onehot, centroids = pl.pallas_call(
        _kmeans_kernel,
        in_specs=[
            pl.BlockSpec(memory_space=pltpu.SMEM),
            pl.BlockSpec(memory_space=pltpu.VMEM),
            pl.BlockSpec(memory_space=pltpu.VMEM),
        ],
        out_specs=[
            pl.BlockSpec(memory_space=pltpu.VMEM),
            pl.BlockSpec(memory_space=pltpu.VMEM),
        ],
        out_shape=[
            jax.ShapeDtypeStruct((_N, _K), jnp.float32),
            jax.ShapeDtypeStruct((_K, _D), jnp.float32),
        ],
    )(it, data, c0)
    return onehot, centroids


# speedup vs baseline: 21.2331x; 1.4706x over previous
import numpy as np
import jax
import jax.numpy as jnp
from jax.experimental import pallas as pl
from jax.experimental.pallas import tpu as pltpu

_N = 4096
_D = 64
_K = 512
_RB = 512
_NB = _N // _RB


def _init_centroid_ids():
    rng = np.random.default_rng(0)
    return np.asarray(rng.choice(_N, size=_K, replace=False))


def _split3(x):
    hi = x.astype(jnp.bfloat16)
    r1 = x - hi.astype(jnp.float32)
    mid = r1.astype(jnp.bfloat16)
    lo = (r1 - mid.astype(jnp.float32)).astype(jnp.bfloat16)
    return hi, mid, lo


def _kmeans_kernel(it_ref, data_ref, c0_ref, oh_ref, cent_ref, xcat_ref):
    iota_k = jax.lax.broadcasted_iota(jnp.int32, (_RB, _K), 1)

    def stage(b, _):
        x = data_ref[pl.ds(b * _RB, _RB), :]
        xh, xm, xl = _split3(x)
        xcat_ref[pl.ds(b * _RB, _RB), :] = jnp.concatenate(
            [xh, xm, xl, xh, xh, xm], axis=1)
        return 0

    jax.lax.fori_loop(0, _NB, stage, 0)

    def prep(c):
        ch, cm, cl = _split3(c)
        ccat = jnp.concatenate([ch, cm, ch, cm, cl, ch], axis=1)
        csq = jnp.sum(c * c, axis=1)[None, :]
        return ccat, csq

    def assign_block(b, ccat, csq):
        xcat = xcat_ref[pl.ds(b * _RB, _RB), :]
        xc = jax.lax.dot_general(
            xcat, ccat, (((1,), (1,)), ((), ())),
            preferred_element_type=jnp.float32)
        x = data_ref[pl.ds(b * _RB, _RB), :]
        xsq = jnp.sum(x * x, axis=1, keepdims=True)
        r = (xsq - 2.0 * xc) + csq
        m = jnp.min(r, axis=1, keepdims=True)
        idx = jnp.min(jnp.where(r == m, iota_k, _K), axis=1, keepdims=True)
        oh = (iota_k == idx).astype(jnp.float32)
        return oh

    def stats_pass(c):
        ccat, csq = prep(c)

        def blk(b, carry):
            acc, cnt = carry
            oh = assign_block(b, ccat, csq)
            xupd = xcat_ref[pl.ds(b * _RB, _RB), 0:192]
            acc = acc + jax.lax.dot_general(
                oh.astype(jnp.bfloat16), xupd, (((0,), (0,)), ((), ())),
                preferred_element_type=jnp.float32)
            cnt = cnt + jnp.sum(oh, axis=0)
            return acc, cnt

        acc0 = jnp.zeros((_K, 3 * _D), jnp.float32)
        cnt0 = jnp.zeros((_K,), jnp.float32)
        acc, cnt = jax.lax.fori_loop(0, _NB, blk, (acc0, cnt0))
        pseudo = acc[:, 0:_D] + acc[:, _D:2 * _D] + acc[:, 2 * _D:3 * _D]
        return pseudo, cnt

    def iter_body(_, c):
        pseudo, cnt = stats_pass(c)
        return pseudo / cnt[:, None]

    c_final = jax.lax.fori_loop(0, it_ref[0], iter_body, c0_ref[...])
    cent_ref[...] = c_final

    ccat, csq = prep(c_final)

    def final_blk(b, _):
        oh_ref[pl.ds(b * _RB, _RB), :] = assign_block(b, ccat, csq)
        return 0

    jax.lax.fori_loop(0, _NB, final_blk, 0)


def kernel(data, iteration):
    c0 = jnp.take(data, jnp.asarray(_init_centroid_ids()), axis=0)
    it = jnp.asarray(iteration, jnp.int32).reshape(1)
    onehot, centroids = pl.pallas_call(
        _kmeans_kernel,
        in_specs=[
            pl.BlockSpec(memory_space=pltpu.SMEM),
            pl.BlockSpec(memory_space=pltpu.VMEM),
            pl.BlockSpec(memory_space=pltpu.VMEM),
        ],
        out_specs=[
            pl.BlockSpec(memory_space=pltpu.VMEM),
            pl.BlockSpec(memory_space=pltpu.VMEM),
        ],
        out_shape=[
            jax.ShapeDtypeStruct((_N, _K), jnp.float32),
            jax.ShapeDtypeStruct((_K, _D), jnp.float32),
        ],
        scratch_shapes=[pltpu.VMEM((_N, 6 * _D), jnp.bfloat16)],
    )(it, data, c0)
    return onehot, centroids


# speedup vs baseline: 24.6666x; 1.1617x over previous
import numpy as np
import jax
import jax.numpy as jnp
from jax.experimental import pallas as pl
from jax.experimental.pallas import tpu as pltpu

_N = 4096
_D = 64
_K = 512
_RB = 512
_NB = _N // _RB


def _init_centroid_ids():
    rng = np.random.default_rng(0)
    return np.asarray(rng.choice(_N, size=_K, replace=False))


def _split3(x):
    hi = x.astype(jnp.bfloat16)
    r1 = x - hi.astype(jnp.float32)
    mid = r1.astype(jnp.bfloat16)
    lo = (r1 - mid.astype(jnp.float32)).astype(jnp.bfloat16)
    return hi, mid, lo


def _kmeans_kernel(it_ref, data_ref, c0_ref, oh_ref, cent_ref, xcat_ref):
    iota_k = jax.lax.broadcasted_iota(jnp.int32, (_RB, _K), 1)

    def stage(b, _):
        x = data_ref[pl.ds(b * _RB, _RB), :]
        xh, xm, xl = _split3(x)
        xcat_ref[pl.ds(b * _RB, _RB), :] = jnp.concatenate(
            [xh, xm, xl, xh, xh, xm], axis=1)
        return 0

    jax.lax.fori_loop(0, _NB, stage, 0)

    def prep(c):
        ch, cm, cl = _split3(c)
        ccat = jnp.concatenate([ch, cm, ch, cm, cl, ch], axis=1)
        csq = jnp.sum(c * c, axis=1)[None, :]
        return ccat, csq

    def assign_block(b, ccat, csq):
        xcat = xcat_ref[pl.ds(b * _RB, _RB), :]
        xc = jax.lax.dot_general(
            xcat, ccat, (((1,), (1,)), ((), ())),
            preferred_element_type=jnp.float32)
        x = data_ref[pl.ds(b * _RB, _RB), :]
        xsq = jnp.sum(x * x, axis=1, keepdims=True)
        r = (xsq - 2.0 * xc) + csq
        m = jnp.min(r, axis=1, keepdims=True)
        idx = jnp.min(jnp.where(r == m, iota_k, _K), axis=1, keepdims=True)
        oh = (iota_k == idx).astype(jnp.float32)
        return oh

    def stats_pass(c):
        ccat, csq = prep(c)

        def blk(b2, carry):
            acc, cnt = carry
            for u in range(2):
                b = b2 * 2 + u
                oh = assign_block(b, ccat, csq)
                xupd = xcat_ref[pl.ds(b * _RB, _RB), 0:192]
                acc = acc + jax.lax.dot_general(
                    oh.astype(jnp.bfloat16), xupd, (((0,), (0,)), ((), ())),
                    preferred_element_type=jnp.float32)
                cnt = cnt + jnp.sum(oh, axis=0)
            return acc, cnt

        acc0 = jnp.zeros((_K, 3 * _D), jnp.float32)
        cnt0 = jnp.zeros((_K,), jnp.float32)
        acc, cnt = jax.lax.fori_loop(0, _NB // 2, blk, (acc0, cnt0))
        pseudo = acc[:, 0:_D] + acc[:, _D:2 * _D] + acc[:, 2 * _D:3 * _D]
        return pseudo, cnt

    def iter_body(_, c):
        pseudo, cnt = stats_pass(c)
        return pseudo / cnt[:, None]

    c_final = jax.lax.fori_loop(0, it_ref[0], iter_body, c0_ref[...])
    cent_ref[...] = c_final

    ccat, csq = prep(c_final)

    def final_blk(b, _):
        oh_ref[pl.ds(b * _RB, _RB), :] = assign_block(b, ccat, csq)
        return 0

    jax.lax.fori_loop(0, _NB, final_blk, 0)


def kernel(data, iteration):
    c0 = jnp.take(data, jnp.asarray(_init_centroid_ids()), axis=0)
    it = jnp.asarray(iteration, jnp.int32).reshape(1)
    onehot, centroids = pl.pallas_call(
        _kmeans_kernel,
        in_specs=[
            pl.BlockSpec(memory_space=pltpu.SMEM),
            pl.BlockSpec(memory_space=pltpu.VMEM),
            pl.BlockSpec(memory_space=pltpu.VMEM),
        ],
        out_specs=[
            pl.BlockSpec(memory_space=pltpu.VMEM),
            pl.BlockSpec(memory_space=pltpu.VMEM),
        ],
        out_shape=[
            jax.ShapeDtypeStruct((_N, _K), jnp.float32),
            jax.ShapeDtypeStruct((_K, _D), jnp.float32),
        ],
        scratch_shapes=[pltpu.VMEM((_N, 6 * _D), jnp.bfloat16)],
    )(it, data, c0)
    return onehot, centroids
